# baseline (speedup 1.0000x reference)
import functools

import jax
import jax.numpy as jnp
from jax import lax
from jax.experimental import pallas as pl
from jax.experimental.pallas import tpu as pltpu
from jax.experimental.pallas import tpu_sc as plsc

BATCH = 16384
EMBED_DIM = 32

try:
    _info = plsc.get_sparse_core_info()
    _NC = _info.num_cores
    _NS = _info.num_subcores
except Exception:
    _NC, _NS = 2, 16
_NW = _NC * _NS
_BPW = BATCH // _NW
_CHUNK = 128
_NCHUNK = _BPW // _CHUNK

_MV_BLK = 65536


def _mv_body(t_ref, w_ref, o_ref):
    o_ref[...] = lax.dot_general(
        w_ref[...], t_ref[...],
        dimension_numbers=(((0,), (0,)), ((), ())),
        preferred_element_type=jnp.float32,
    )[0]


def _matvec(t_t, w):
    d, n = t_t.shape
    grid = (n + _MV_BLK - 1) // _MV_BLK
    return pl.pallas_call(
        _mv_body,
        grid=(grid,),
        in_specs=[
            pl.BlockSpec((d, _MV_BLK), lambda i: (0, i)),
            pl.BlockSpec((d, 1), lambda i: (0, 0)),
        ],
        out_specs=pl.BlockSpec((_MV_BLK,), lambda i: (i,)),
        out_shape=jax.ShapeDtypeStruct((n,), jnp.float32),
    )(t_t, w)


_MT_N = 100000
_MT_ALIGNED = (_MT_N // 128) * 128
_MT_W0 = 3200
_MT_W1 = 3072
_MT_SPLIT = 13


def _scmv_body(t_hbm, wb_hbm, o_hbm, tva, tvb, wbv, ov, sema, semb):
    wid = lax.axis_index("s") * _NC + lax.axis_index("c")
    pltpu.sync_copy(wb_hbm, wbv)
    wvecs = [wbv[pl.ds(d * 16, 16)] for d in range(EMBED_DIM)]

    def strip(nch, col0):
        def start(g, tv, sem):
            pltpu.async_copy(
                t_hbm.at[:, pl.ds(col0 + g * 128, 128)], tv, sem)

        def wait(tv, sem):
            pltpu.make_async_copy(
                t_hbm.at[:, pl.ds(col0, 128)], tv, sem).wait()

        def compute(tv, g):
            for c in range(8):
                acc = wvecs[0] * tv[0, pl.ds(c * 16, 16)]
                for d in range(1, EMBED_DIM):
                    acc = acc + wvecs[d] * tv[d, pl.ds(c * 16, 16)]
                ov[pl.ds(g * 128 + c * 16, 16)] = acc

        start(0, tva, sema)

        def body(i, carry):
            g0 = i * 2
            g1 = g0 + 1

            @pl.when(g1 < nch)
            def _():
                start(g1, tvb, semb)

            wait(tva, sema)
            compute(tva, g0)

            @pl.when(g0 + 2 < nch)
            def _():
                start(g0 + 2, tva, sema)

            @pl.when(g1 < nch)
            def _():
                wait(tvb, semb)
                compute(tvb, g1)

            return carry

        lax.fori_loop(0, (nch + 1) // 2, body, 0)
        pltpu.sync_copy(ov.at[pl.ds(0, nch * 128)],
                        o_hbm.at[pl.ds(col0, nch * 128)])

    @pl.when(wid < _MT_SPLIT)
    def _():
        strip(_MT_W0 // 128, wid * _MT_W0)

    @pl.when(wid >= _MT_SPLIT)
    def _():
        strip(_MT_W1 // 128, _MT_SPLIT * _MT_W0 + (wid - _MT_SPLIT) * _MT_W1)


def _sc_matvec_movie(mt_t, wb):
    m = functools.partial(
        pl.kernel,
        mesh=plsc.VectorSubcoreMesh(core_axis_name="c", subcore_axis_name="s"),
        out_type=jax.ShapeDtypeStruct((_MT_ALIGNED,), jnp.float32),
        compiler_params=pltpu.CompilerParams(use_tc_tiling_on_sc=True),
        scratch_types=[
            pltpu.VMEM((EMBED_DIM, 128), jnp.float32),
            pltpu.VMEM((EMBED_DIM, 128), jnp.float32),
            pltpu.VMEM((EMBED_DIM * 16,), jnp.float32),
            pltpu.VMEM((_MT_W0,), jnp.float32),
            pltpu.SemaphoreType.DMA,
            pltpu.SemaphoreType.DMA,
        ],
    )(_scmv_body)
    return m(mt_t, wb)


def _sc_body(uid_hbm, mid_hbm, udot_hbm, mdot_hbm, b_hbm, out_hbm,
             uidx, midx, uval, mval, bv, outv, sem):
    wid = lax.axis_index("s") * _NC + lax.axis_index("c")
    base = wid * _BPW

    pltpu.sync_copy(uid_hbm.at[wid], uidx)
    pltpu.sync_copy(mid_hbm.at[wid], midx)
    pltpu.sync_copy(b_hbm, bv)

    copies = []
    for c in range(_NCHUNK):
        copies.append(pltpu.async_copy(udot_hbm.at[uidx.at[c]], uval.at[c], sem))
        copies.append(pltpu.async_copy(mdot_hbm.at[midx.at[c]], mval.at[c], sem))
    for cp in copies:
        cp.wait()

    bvec = bv[...]
    for c in range(_NCHUNK):
        for k in range(_CHUNK // 16):
            v = uval[c, pl.ds(k * 16, 16)] + mval[c, pl.ds(k * 16, 16)] + bvec
            outv[pl.ds(c * _CHUNK + k * 16, 16)] = v

    pltpu.sync_copy(outv, out_hbm.at[pl.ds(base, _BPW)])


@jax.jit
def _run(user_ids, movie_ids, user_table, movie_table, fc_w, fc_b):
    udot = _matvec(user_table.T, fc_w[:EMBED_DIM])
    wb_m = jnp.broadcast_to(fc_w[EMBED_DIM:], (EMBED_DIM, 16)).reshape(-1)
    mt_t = movie_table.T
    mdot_main = _sc_matvec_movie(mt_t, wb_m)
    mdot_tail = _matvec(mt_t[:, _MT_ALIGNED:], fc_w[EMBED_DIM:])
    mdot = jnp.concatenate([mdot_main, mdot_tail])
    uid3d = user_ids.astype(jnp.int32).reshape(_NW, _NCHUNK, _CHUNK)
    mid3d = movie_ids.astype(jnp.int32).reshape(_NW, _NCHUNK, _CHUNK)
    bias16 = jnp.broadcast_to(fc_b.reshape(()), (16,))

    g = functools.partial(
        pl.kernel,
        mesh=plsc.VectorSubcoreMesh(core_axis_name="c", subcore_axis_name="s"),
        out_type=jax.ShapeDtypeStruct((BATCH,), jnp.float32),
        compiler_params=pltpu.CompilerParams(
            needs_layout_passes=False, use_tc_tiling_on_sc=False),
        scratch_types=[
            pltpu.VMEM((_NCHUNK, _CHUNK), jnp.int32),
            pltpu.VMEM((_NCHUNK, _CHUNK), jnp.int32),
            pltpu.VMEM((_NCHUNK, _CHUNK), jnp.float32),
            pltpu.VMEM((_NCHUNK, _CHUNK), jnp.float32),
            pltpu.VMEM((16,), jnp.float32),
            pltpu.VMEM((_BPW,), jnp.float32),
            pltpu.SemaphoreType.DMA,
        ],
    )(_sc_body)
    return g(uid3d, mid3d, udot, mdot, bias16)


def kernel(user_ids, movie_ids, user_table, movie_table, fc_w, fc_b):
    return _run(user_ids, movie_ids, user_table, movie_table, fc_w, fc_b)


# speedup vs baseline: 1.0317x; 1.0317x over previous
import functools

import jax
import jax.numpy as jnp
from jax import lax
from jax.experimental import pallas as pl
from jax.experimental.pallas import tpu as pltpu
from jax.experimental.pallas import tpu_sc as plsc

BATCH = 16384
EMBED_DIM = 32

try:
    _info = plsc.get_sparse_core_info()
    _NC = _info.num_cores
    _NS = _info.num_subcores
except Exception:
    _NC, _NS = 2, 16
_NW = _NC * _NS
_BPW = BATCH // _NW
_CHUNK = 128
_NCHUNK = _BPW // _CHUNK

_MV_BLK = 65536


def _mv_body(t_ref, w_ref, o_ref):
    o_ref[...] = lax.dot_general(
        w_ref[...], t_ref[...],
        dimension_numbers=(((0,), (0,)), ((), ())),
        preferred_element_type=jnp.float32,
    )[0]


def _matvec(t_t, w):
    d, n = t_t.shape
    grid = (n + _MV_BLK - 1) // _MV_BLK
    return pl.pallas_call(
        _mv_body,
        grid=(grid,),
        in_specs=[
            pl.BlockSpec((d, _MV_BLK), lambda i: (0, i)),
            pl.BlockSpec((d, 1), lambda i: (0, 0)),
        ],
        out_specs=pl.BlockSpec((_MV_BLK,), lambda i: (i,)),
        out_shape=jax.ShapeDtypeStruct((n,), jnp.float32),
    )(t_t, w)


_MT_N = 100000
_MT_ALIGNED = (_MT_N // 128) * 128
_MT_W0 = 3200
_MT_W1 = 3072
_MT_SPLIT = 13


def _scmv_body(t_hbm, wb_hbm, o_hbm, tv, wbv, ov, sema):
    wid = lax.axis_index("s") * _NC + lax.axis_index("c")
    pltpu.sync_copy(wb_hbm, wbv)

    def strip(w, col0):
        for s in range(4):
            pltpu.async_copy(
                t_hbm.at[pl.ds(8 * s, 8), pl.ds(col0, w)],
                tv.at[pl.ds(8 * s, 8), pl.ds(0, w)], sema)
        for s in range(4):
            pltpu.make_async_copy(
                t_hbm.at[pl.ds(0, 8), pl.ds(col0, w)],
                tv.at[pl.ds(0, 8), pl.ds(0, w)], sema).wait()

        def body(i, carry):
            c0 = i * 16
            acc = wbv[pl.ds(0, 16)] * tv[0, pl.ds(c0, 16)]
            for d in range(1, EMBED_DIM):
                acc = acc + wbv[pl.ds(d * 16, 16)] * tv[d, pl.ds(c0, 16)]
            ov[pl.ds(c0, 16)] = acc
            return carry

        lax.fori_loop(0, w // 16, body, 0)
        pltpu.sync_copy(ov.at[pl.ds(0, w)], o_hbm.at[pl.ds(col0, w)])

    @pl.when(wid < _MT_SPLIT)
    def _():
        strip(_MT_W0, wid * _MT_W0)

    @pl.when(wid >= _MT_SPLIT)
    def _():
        strip(_MT_W1, _MT_SPLIT * _MT_W0 + (wid - _MT_SPLIT) * _MT_W1)


def _sc_matvec_movie(mt_t, wb):
    m = functools.partial(
        pl.kernel,
        mesh=plsc.VectorSubcoreMesh(core_axis_name="c", subcore_axis_name="s"),
        out_type=jax.ShapeDtypeStruct((_MT_ALIGNED,), jnp.float32),
        compiler_params=pltpu.CompilerParams(use_tc_tiling_on_sc=True),
        scratch_types=[
            pltpu.VMEM((EMBED_DIM, _MT_W0), jnp.float32),
            pltpu.VMEM((EMBED_DIM * 16,), jnp.float32),
            pltpu.VMEM((_MT_W0,), jnp.float32),
            pltpu.SemaphoreType.DMA,
        ],
    )(_scmv_body)
    return m(mt_t, wb)


def _sc_body(uid_hbm, mid_hbm, udot_hbm, mdot_hbm, b_hbm, out_hbm,
             uidx, midx, uval, mval, bv, outv, sem):
    wid = lax.axis_index("s") * _NC + lax.axis_index("c")
    base = wid * _BPW

    pltpu.sync_copy(uid_hbm.at[wid], uidx)
    pltpu.sync_copy(mid_hbm.at[wid], midx)
    pltpu.sync_copy(b_hbm, bv)

    copies = []
    for c in range(_NCHUNK):
        copies.append(pltpu.async_copy(udot_hbm.at[uidx.at[c]], uval.at[c], sem))
        copies.append(pltpu.async_copy(mdot_hbm.at[midx.at[c]], mval.at[c], sem))
    for cp in copies:
        cp.wait()

    bvec = bv[...]
    for c in range(_NCHUNK):
        for k in range(_CHUNK // 16):
            v = uval[c, pl.ds(k * 16, 16)] + mval[c, pl.ds(k * 16, 16)] + bvec
            outv[pl.ds(c * _CHUNK + k * 16, 16)] = v

    pltpu.sync_copy(outv, out_hbm.at[pl.ds(base, _BPW)])


@jax.jit
def _run(user_ids, movie_ids, user_table, movie_table, fc_w, fc_b):
    udot = _matvec(user_table.T, fc_w[:EMBED_DIM])
    wb_m = jnp.broadcast_to(fc_w[EMBED_DIM:], (EMBED_DIM, 16)).reshape(-1)
    mt_t = movie_table.T
    mdot_main = _sc_matvec_movie(mt_t, wb_m)
    mdot_tail = _matvec(mt_t[:, _MT_ALIGNED:], fc_w[EMBED_DIM:])
    mdot = jnp.concatenate([mdot_main, mdot_tail])
    uid3d = user_ids.astype(jnp.int32).reshape(_NW, _NCHUNK, _CHUNK)
    mid3d = movie_ids.astype(jnp.int32).reshape(_NW, _NCHUNK, _CHUNK)
    bias16 = jnp.broadcast_to(fc_b.reshape(()), (16,))

    g = functools.partial(
        pl.kernel,
        mesh=plsc.VectorSubcoreMesh(core_axis_name="c", subcore_axis_name="s"),
        out_type=jax.ShapeDtypeStruct((BATCH,), jnp.float32),
        compiler_params=pltpu.CompilerParams(
            needs_layout_passes=False, use_tc_tiling_on_sc=False),
        scratch_types=[
            pltpu.VMEM((_NCHUNK, _CHUNK), jnp.int32),
            pltpu.VMEM((_NCHUNK, _CHUNK), jnp.int32),
            pltpu.VMEM((_NCHUNK, _CHUNK), jnp.float32),
            pltpu.VMEM((_NCHUNK, _CHUNK), jnp.float32),
            pltpu.VMEM((16,), jnp.float32),
            pltpu.VMEM((_BPW,), jnp.float32),
            pltpu.SemaphoreType.DMA,
        ],
    )(_sc_body)
    return g(uid3d, mid3d, udot, mdot, bias16)


def kernel(user_ids, movie_ids, user_table, movie_table, fc_w, fc_b):
    return _run(user_ids, movie_ids, user_table, movie_table, fc_w, fc_b)


# speedup vs baseline: 1.0911x; 1.0576x over previous
import functools

import jax
import jax.numpy as jnp
from jax import lax
from jax.experimental import pallas as pl
from jax.experimental.pallas import tpu as pltpu
from jax.experimental.pallas import tpu_sc as plsc

BATCH = 16384
EMBED_DIM = 32

try:
    _info = plsc.get_sparse_core_info()
    _NC = _info.num_cores
    _NS = _info.num_subcores
except Exception:
    _NC, _NS = 2, 16
_NW = _NC * _NS
_BPW = BATCH // _NW
_CHUNK = 128
_NCHUNK = _BPW // _CHUNK

_MV_BLK = 65536


def _mv_body(t_ref, w_ref, o_ref):
    o_ref[...] = lax.dot_general(
        w_ref[...], t_ref[...],
        dimension_numbers=(((0,), (0,)), ((), ())),
        preferred_element_type=jnp.float32,
    )[0]


def _matvec(t_t, w):
    d, n = t_t.shape
    grid = (n + _MV_BLK - 1) // _MV_BLK
    return pl.pallas_call(
        _mv_body,
        grid=(grid,),
        in_specs=[
            pl.BlockSpec((d, _MV_BLK), lambda i: (0, i)),
            pl.BlockSpec((d, 1), lambda i: (0, 0)),
        ],
        out_specs=pl.BlockSpec((_MV_BLK,), lambda i: (i,)),
        out_shape=jax.ShapeDtypeStruct((n,), jnp.float32),
    )(t_t, w)


def _mv2_body(u_ref, m_ref, wu_ref, wm_ref, ou_ref, om_ref):
    i = pl.program_id(0)
    nu = pl.num_programs(0) - 1

    @pl.when(i < nu)
    def _():
        ou_ref[...] = lax.dot_general(
            wu_ref[...], u_ref[...],
            dimension_numbers=(((0,), (0,)), ((), ())),
            preferred_element_type=jnp.float32,
        )[0]

    @pl.when(i == nu)
    def _():
        om_ref[...] = lax.dot_general(
            wm_ref[...], m_ref[...],
            dimension_numbers=(((0,), (0,)), ((), ())),
            preferred_element_type=jnp.float32,
        )[0]


def _matvec2(ut_t, mt_t, wu, wm):
    d, nu = ut_t.shape
    _, nm = mt_t.shape
    gu = (nu + _MV_BLK - 1) // _MV_BLK
    mblk = ((nm + 1023) // 1024) * 1024
    return pl.pallas_call(
        _mv2_body,
        grid=(gu + 1,),
        in_specs=[
            pl.BlockSpec((d, _MV_BLK), lambda i: (0, jnp.minimum(i, gu - 1))),
            pl.BlockSpec((d, mblk), lambda i: (0, 0)),
            pl.BlockSpec((d, 1), lambda i: (0, 0)),
            pl.BlockSpec((d, 1), lambda i: (0, 0)),
        ],
        out_specs=[
            pl.BlockSpec((_MV_BLK,), lambda i: (jnp.minimum(i, gu - 1),)),
            pl.BlockSpec((mblk,), lambda i: (0,)),
        ],
        out_shape=[
            jax.ShapeDtypeStruct((nu,), jnp.float32),
            jax.ShapeDtypeStruct((nm,), jnp.float32),
        ],
    )(ut_t, mt_t, wu, wm)


_MT_N = 100000
_MT_ALIGNED = (_MT_N // 128) * 128
_MT_W0 = 3200
_MT_W1 = 3072
_MT_SPLIT = 13


def _scmv_body(t_hbm, wb_hbm, o_hbm, tv, wbv, ov, sema):
    wid = lax.axis_index("s") * _NC + lax.axis_index("c")
    pltpu.sync_copy(wb_hbm, wbv)

    def strip(w, col0):
        for s in range(4):
            pltpu.async_copy(
                t_hbm.at[pl.ds(8 * s, 8), pl.ds(col0, w)],
                tv.at[pl.ds(8 * s, 8), pl.ds(0, w)], sema)
        for s in range(4):
            pltpu.make_async_copy(
                t_hbm.at[pl.ds(0, 8), pl.ds(col0, w)],
                tv.at[pl.ds(0, 8), pl.ds(0, w)], sema).wait()

        def body(i, carry):
            c0 = i * 16
            acc = wbv[pl.ds(0, 16)] * tv[0, pl.ds(c0, 16)]
            for d in range(1, EMBED_DIM):
                acc = acc + wbv[pl.ds(d * 16, 16)] * tv[d, pl.ds(c0, 16)]
            ov[pl.ds(c0, 16)] = acc
            return carry

        lax.fori_loop(0, w // 16, body, 0)
        pltpu.sync_copy(ov.at[pl.ds(0, w)], o_hbm.at[pl.ds(col0, w)])

    @pl.when(wid < _MT_SPLIT)
    def _():
        strip(_MT_W0, wid * _MT_W0)

    @pl.when(wid >= _MT_SPLIT)
    def _():
        strip(_MT_W1, _MT_SPLIT * _MT_W0 + (wid - _MT_SPLIT) * _MT_W1)


def _sc_matvec_movie(mt_t, wb):
    m = functools.partial(
        pl.kernel,
        mesh=plsc.VectorSubcoreMesh(core_axis_name="c", subcore_axis_name="s"),
        out_type=jax.ShapeDtypeStruct((_MT_ALIGNED,), jnp.float32),
        compiler_params=pltpu.CompilerParams(use_tc_tiling_on_sc=True),
        scratch_types=[
            pltpu.VMEM((EMBED_DIM, _MT_W0), jnp.float32),
            pltpu.VMEM((EMBED_DIM * 16,), jnp.float32),
            pltpu.VMEM((_MT_W0,), jnp.float32),
            pltpu.SemaphoreType.DMA,
        ],
    )(_scmv_body)
    return m(mt_t, wb)


def _sc_body(uid_hbm, mid_hbm, udot_hbm, mdot_hbm, b_hbm, out_hbm,
             uidx, midx, uval, mval, bv, outv, sem):
    wid = lax.axis_index("s") * _NC + lax.axis_index("c")
    base = wid * _BPW

    pltpu.sync_copy(uid_hbm.at[wid], uidx)
    pltpu.sync_copy(mid_hbm.at[wid], midx)
    pltpu.sync_copy(b_hbm, bv)

    copies = []
    for c in range(_NCHUNK):
        copies.append(pltpu.async_copy(udot_hbm.at[uidx.at[c]], uval.at[c], sem))
        copies.append(pltpu.async_copy(mdot_hbm.at[midx.at[c]], mval.at[c], sem))
    for cp in copies:
        cp.wait()

    bvec = bv[...]
    for c in range(_NCHUNK):
        for k in range(_CHUNK // 16):
            v = uval[c, pl.ds(k * 16, 16)] + mval[c, pl.ds(k * 16, 16)] + bvec
            outv[pl.ds(c * _CHUNK + k * 16, 16)] = v

    pltpu.sync_copy(outv, out_hbm.at[pl.ds(base, _BPW)])


@jax.jit
def _run(user_ids, movie_ids, user_table, movie_table, fc_w, fc_b):
    udot, mdot = _matvec2(user_table.T, movie_table.T,
                          fc_w[:EMBED_DIM], fc_w[EMBED_DIM:])
    uid3d = user_ids.astype(jnp.int32).reshape(_NW, _NCHUNK, _CHUNK)
    mid3d = movie_ids.astype(jnp.int32).reshape(_NW, _NCHUNK, _CHUNK)
    bias16 = jnp.broadcast_to(fc_b.reshape(()), (16,))

    g = functools.partial(
        pl.kernel,
        mesh=plsc.VectorSubcoreMesh(core_axis_name="c", subcore_axis_name="s"),
        out_type=jax.ShapeDtypeStruct((BATCH,), jnp.float32),
        compiler_params=pltpu.CompilerParams(
            needs_layout_passes=False, use_tc_tiling_on_sc=False),
        scratch_types=[
            pltpu.VMEM((_NCHUNK, _CHUNK), jnp.int32),
            pltpu.VMEM((_NCHUNK, _CHUNK), jnp.int32),
            pltpu.VMEM((_NCHUNK, _CHUNK), jnp.float32),
            pltpu.VMEM((_NCHUNK, _CHUNK), jnp.float32),
            pltpu.VMEM((16,), jnp.float32),
            pltpu.VMEM((_BPW,), jnp.float32),
            pltpu.SemaphoreType.DMA,
        ],
    )(_sc_body)
    return g(uid3d, mid3d, udot, mdot, bias16)


def kernel(user_ids, movie_ids, user_table, movie_table, fc_w, fc_b):
    return _run(user_ids, movie_ids, user_table, movie_table, fc_w, fc_b)
